# baseline (speedup 1.0000x reference)
import functools

import jax
import jax.numpy as jnp
from jax import lax
from jax.experimental import pallas as pl
from jax.experimental.pallas import tpu as pltpu
from jax.experimental.pallas import tpu_sc as plsc

_VOCAB = 1000
_EMBED = 64
_NW = 32
_LANES = 16
_RUN = 64


def _make_sc_gather(seq_len, batch):
    half = batch // 2
    nhb = half // _RUN
    n_units = seq_len * nhb
    upw = n_units // _NW
    mesh = plsc.VectorSubcoreMesh(core_axis_name="c", subcore_axis_name="s")

    @functools.partial(
        pl.kernel,
        mesh=mesh,
        compiler_params=pltpu.CompilerParams(use_tc_tiling_on_sc=False),
        out_type=jax.ShapeDtypeStruct((seq_len, half, 2 * _EMBED), jnp.bfloat16),
        scratch_types=[
            pltpu.VMEM((8, 1024), jnp.int32),
            pltpu.VMEM((2, 2 * _RUN), jnp.int32),
            pltpu.VMEM((2, 2, _RUN, 2 * _EMBED), jnp.bfloat16),
            pltpu.SemaphoreType.DMA,
            pltpu.SemaphoreType.DMA,
        ],
    )
    def sc_gather(xt_hbm, table_hbm, out_hbm, tok_v, keys_v, rows_v,
                  sem0, sem1):
        wid = lax.axis_index("s") * 2 + lax.axis_index("c")
        sems = (sem0, sem1)
        l0 = (wid * upw) // nhb
        lstart = jnp.maximum(l0 - 1, 0)
        pltpu.sync_copy(xt_hbm.at[pl.ds(lstart, 8)], tok_v)

        def unit(i):
            u = wid * upw + i
            return u // nhb, (u % nhb) * _RUN

        def stage_fire(i, slot):
            l, bb = unit(i)
            li = l - lstart
            lp = jnp.maximum(li - 1, 0)
            for side in range(2):
                off = side * half
                for j in range(_RUN // _LANES):
                    cur = tok_v[li, pl.ds(off + bb + j * _LANES, _LANES)]
                    prv = tok_v[lp, pl.ds(off + bb + j * _LANES, _LANES)]
                    keys_v[slot, pl.ds(side * _RUN + j * _LANES, _LANES)] = (
                        jnp.where(l > 0, prv * _VOCAB + cur, cur))
            pltpu.async_copy(table_hbm.at[keys_v.at[slot, pl.ds(0, _RUN)]],
                             rows_v.at[slot, 0], sems[slot])
            pltpu.async_copy(table_hbm.at[keys_v.at[slot, pl.ds(_RUN, _RUN)]],
                             rows_v.at[slot, 1], sems[slot])

        def drain(i, slot):
            pltpu.make_async_copy(
                table_hbm.at[keys_v.at[slot, pl.ds(0, _RUN)]],
                rows_v.at[slot, 0], sems[slot]).wait()
            pltpu.make_async_copy(
                table_hbm.at[keys_v.at[slot, pl.ds(_RUN, _RUN)]],
                rows_v.at[slot, 1], sems[slot]).wait()

        def scat(i, slot):
            l, bb = unit(i)
            pltpu.sync_copy(rows_v.at[slot, 0, slice(None), pl.ds(0, _EMBED)],
                            out_hbm.at[l, pl.ds(bb, _RUN), pl.ds(0, _EMBED)])
            pltpu.sync_copy(rows_v.at[slot, 1, slice(None), pl.ds(0, _EMBED)],
                            out_hbm.at[l, pl.ds(bb, _RUN), pl.ds(_EMBED, _EMBED)])

        stage_fire(0, 0)

        def body(i2, carry):
            k0 = i2 * 2
            stage_fire(k0 + 1, 1)
            drain(k0, 0)
            scat(k0, 0)

            @pl.when(k0 + 2 < upw)
            def _():
                stage_fire(k0 + 2, 0)

            drain(k0 + 1, 1)
            scat(k0 + 1, 1)
            return carry

        lax.fori_loop(0, upw // 2, body, 0)

    return sc_gather


_TCB = 2048


def _tpad_body(tin_ref, tout_ref):
    t = jnp.swapaxes(tin_ref[...], 0, 1).astype(jnp.bfloat16)
    tout_ref[...] = jnp.concatenate(
        [t, jnp.zeros((_TCB, _EMBED), jnp.bfloat16)], axis=1)


def _transpose_pad_table(table_t):
    nrows = table_t.shape[1]
    return pl.pallas_call(
        _tpad_body,
        grid=((nrows + _TCB - 1) // _TCB,),
        in_specs=[pl.BlockSpec((_EMBED, _TCB), lambda i: (0, i))],
        out_specs=pl.BlockSpec((_TCB, 2 * _EMBED), lambda i: (i, 0)),
        out_shape=jax.ShapeDtypeStruct((nrows, 2 * _EMBED), jnp.bfloat16),
    )(table_t)


def _dense_body(h_ref, m_ref, gw_ref, gb_ref, ow_ref, ob_ref, out_ref, gate_ref):
    h = h_ref[0]
    mp = m_ref[0]
    dn = (((0,), (1,)), ((), ()))
    yl = lax.dot_general(ow_ref[...], mp[:, :_EMBED].astype(jnp.float32), dn,
                         preferred_element_type=jnp.float32)
    yr = lax.dot_general(ow_ref[...], mp[:, _EMBED:].astype(jnp.float32), dn,
                         preferred_element_type=jnp.float32)
    y = jnp.concatenate([yl, yr], axis=1) + ob_ref[...]
    g = jax.nn.sigmoid(
        jnp.sum(h * gw_ref[...], axis=0, keepdims=True) + gb_ref[0, 0])
    out_ref[0] = h + g * y
    gate_ref[0] = g


def kernel(x_tokens, hidden_state, memory_table, gate_w, gate_b, out_w, out_b):
    b, l = x_tokens.shape
    x_t = jnp.transpose(x_tokens).astype(jnp.int32)
    table128 = _transpose_pad_table(jnp.transpose(memory_table))
    gathered_t = _make_sc_gather(l, b)(x_t, table128)

    h_t = jnp.transpose(hidden_state, (1, 2, 0))
    out_t, gate_t = pl.pallas_call(
        _dense_body,
        grid=(l,),
        in_specs=[
            pl.BlockSpec((1, _EMBED, b), lambda i: (i, 0, 0)),
            pl.BlockSpec((1, b // 2, 2 * _EMBED), lambda i: (i, 0, 0)),
            pl.BlockSpec((_EMBED, 1), lambda i: (0, 0)),
            pl.BlockSpec((1, 1), lambda i: (0, 0)),
            pl.BlockSpec((_EMBED, _EMBED), lambda i: (0, 0)),
            pl.BlockSpec((_EMBED, 1), lambda i: (0, 0)),
        ],
        out_specs=[
            pl.BlockSpec((1, _EMBED, b), lambda i: (i, 0, 0)),
            pl.BlockSpec((1, 1, b), lambda i: (i, 0, 0)),
        ],
        out_shape=[
            jax.ShapeDtypeStruct((l, _EMBED, b), jnp.float32),
            jax.ShapeDtypeStruct((l, 1, b), jnp.float32),
        ],
    )(h_t, gathered_t, gate_w, gate_b.reshape(1, 1), out_w,
      out_b.reshape(_EMBED, 1))

    return (jnp.transpose(out_t, (2, 0, 1)),
            jnp.transpose(gate_t, (2, 0, 1)))


# speedup vs baseline: 2.2242x; 2.2242x over previous
import functools

import jax
import jax.numpy as jnp
from jax import lax
from jax.experimental import pallas as pl
from jax.experimental.pallas import tpu as pltpu
from jax.experimental.pallas import tpu_sc as plsc

_VOCAB = 1000
_EMBED = 64
_NW = 32
_LANES = 16
_RUN = 64


def _make_sc_gather(seq_len, batch):
    half = batch // 2
    nhb = half // _RUN
    n_units = seq_len * nhb
    upw = n_units // _NW
    mesh = plsc.VectorSubcoreMesh(core_axis_name="c", subcore_axis_name="s")

    @functools.partial(
        pl.kernel,
        mesh=mesh,
        compiler_params=pltpu.CompilerParams(use_tc_tiling_on_sc=False),
        out_type=jax.ShapeDtypeStruct((seq_len, half, 2 * _EMBED), jnp.float32),
        scratch_types=[
            pltpu.VMEM((8, 1024), jnp.int32),
            pltpu.VMEM((2, 2 * _RUN), jnp.int32),
            pltpu.VMEM((2, 2, _RUN, 2 * _EMBED), jnp.float32),
            pltpu.SemaphoreType.DMA,
            pltpu.SemaphoreType.DMA,
        ],
    )
    def sc_gather(xt_hbm, table_hbm, out_hbm, tok_v, keys_v, rows_v,
                  sem0, sem1):
        wid = lax.axis_index("s") * 2 + lax.axis_index("c")
        sems = (sem0, sem1)
        l0 = (wid * upw) // nhb
        lstart = jnp.maximum(l0 - 1, 0)
        pltpu.sync_copy(xt_hbm.at[pl.ds(lstart, 8)], tok_v)

        def unit(i):
            u = wid * upw + i
            return u // nhb, (u % nhb) * _RUN

        def stage_fire(i, slot):
            l, bb = unit(i)
            li = l - lstart
            lp = jnp.maximum(li - 1, 0)
            for side in range(2):
                off = side * half
                for j in range(_RUN // _LANES):
                    cur = tok_v[li, pl.ds(off + bb + j * _LANES, _LANES)]
                    prv = tok_v[lp, pl.ds(off + bb + j * _LANES, _LANES)]
                    keys_v[slot, pl.ds(side * _RUN + j * _LANES, _LANES)] = (
                        jnp.where(l > 0, prv * _VOCAB + cur, cur))
            pltpu.async_copy(table_hbm.at[keys_v.at[slot, pl.ds(0, _RUN)]],
                             rows_v.at[slot, 0], sems[slot])
            pltpu.async_copy(table_hbm.at[keys_v.at[slot, pl.ds(_RUN, _RUN)]],
                             rows_v.at[slot, 1], sems[slot])

        def drain(i, slot):
            pltpu.make_async_copy(
                table_hbm.at[keys_v.at[slot, pl.ds(0, _RUN)]],
                rows_v.at[slot, 0], sems[slot]).wait()
            pltpu.make_async_copy(
                table_hbm.at[keys_v.at[slot, pl.ds(_RUN, _RUN)]],
                rows_v.at[slot, 1], sems[slot]).wait()

        def scat(i, slot):
            l, bb = unit(i)
            pltpu.sync_copy(rows_v.at[slot, 0, slice(None), pl.ds(0, _EMBED)],
                            out_hbm.at[l, pl.ds(bb, _RUN), pl.ds(0, _EMBED)])
            pltpu.sync_copy(rows_v.at[slot, 1, slice(None), pl.ds(0, _EMBED)],
                            out_hbm.at[l, pl.ds(bb, _RUN), pl.ds(_EMBED, _EMBED)])

        stage_fire(0, 0)

        def body(i2, carry):
            k0 = i2 * 2
            stage_fire(k0 + 1, 1)
            drain(k0, 0)
            scat(k0, 0)

            @pl.when(k0 + 2 < upw)
            def _():
                stage_fire(k0 + 2, 0)

            drain(k0 + 1, 1)
            scat(k0 + 1, 1)
            return carry

        lax.fori_loop(0, upw // 2, body, 0)

    return sc_gather


_TCB = 2048


def _tpad_body(tin_ref, tout_ref):
    t = jnp.swapaxes(tin_ref[...], 0, 1)
    tout_ref[...] = jnp.concatenate(
        [t, jnp.zeros((_TCB, _EMBED), jnp.float32)], axis=1)


def _transpose_pad_table(table_t):
    nrows = table_t.shape[1]
    return pl.pallas_call(
        _tpad_body,
        grid=((nrows + _TCB - 1) // _TCB,),
        in_specs=[pl.BlockSpec((_EMBED, _TCB), lambda i: (0, i))],
        out_specs=pl.BlockSpec((_TCB, 2 * _EMBED), lambda i: (i, 0)),
        out_shape=jax.ShapeDtypeStruct((nrows, 2 * _EMBED), jnp.float32),
    )(table_t)


def _dense_body(h_ref, m_ref, gw_ref, gb_ref, ow_ref, ob_ref, out_ref, gate_ref):
    h = h_ref[0]
    mp = m_ref[0]
    dn = (((0,), (1,)), ((), ()))
    yl = lax.dot_general(ow_ref[...], mp[:, :_EMBED], dn,
                         preferred_element_type=jnp.float32)
    yr = lax.dot_general(ow_ref[...], mp[:, _EMBED:], dn,
                         preferred_element_type=jnp.float32)
    y = jnp.concatenate([yl, yr], axis=1) + ob_ref[...]
    g = jax.nn.sigmoid(
        jnp.sum(h * gw_ref[...], axis=0, keepdims=True) + gb_ref[0, 0])
    out_ref[0] = h + g * y
    gate_ref[0] = g


def kernel(x_tokens, hidden_state, memory_table, gate_w, gate_b, out_w, out_b):
    b, l = x_tokens.shape
    x_t = jnp.transpose(x_tokens).astype(jnp.int32)
    table128 = _transpose_pad_table(jnp.transpose(memory_table))
    gathered_t = _make_sc_gather(l, b)(x_t, table128)

    h_t = jnp.transpose(hidden_state, (1, 2, 0))
    out_t, gate_t = pl.pallas_call(
        _dense_body,
        grid=(l,),
        in_specs=[
            pl.BlockSpec((1, _EMBED, b), lambda i: (i, 0, 0)),
            pl.BlockSpec((1, b // 2, 2 * _EMBED), lambda i: (i, 0, 0)),
            pl.BlockSpec((_EMBED, 1), lambda i: (0, 0)),
            pl.BlockSpec((1, 1), lambda i: (0, 0)),
            pl.BlockSpec((_EMBED, _EMBED), lambda i: (0, 0)),
            pl.BlockSpec((_EMBED, 1), lambda i: (0, 0)),
        ],
        out_specs=[
            pl.BlockSpec((1, _EMBED, b), lambda i: (i, 0, 0)),
            pl.BlockSpec((1, 1, b), lambda i: (i, 0, 0)),
        ],
        out_shape=[
            jax.ShapeDtypeStruct((l, _EMBED, b), jnp.float32),
            jax.ShapeDtypeStruct((l, 1, b), jnp.float32),
        ],
    )(h_t, gathered_t, gate_w, gate_b.reshape(1, 1), out_w,
      out_b.reshape(_EMBED, 1))

    return (jnp.transpose(out_t, (2, 0, 1)),
            jnp.transpose(gate_t, (2, 0, 1)))


# speedup vs baseline: 3.1003x; 1.3939x over previous
import functools

import jax
import jax.numpy as jnp
from jax import lax
from jax.experimental import pallas as pl
from jax.experimental.pallas import tpu as pltpu
from jax.experimental.pallas import tpu_sc as plsc

_VOCAB = 1000
_EMBED = 64
_NW = 32
_LANES = 16
_RUN = 64


def _make_sc_gather(seq_len, batch):
    half = batch // 2
    nhb = half // _RUN
    n_units = seq_len * nhb
    upw = n_units // _NW
    mesh = plsc.VectorSubcoreMesh(core_axis_name="c", subcore_axis_name="s")

    @functools.partial(
        pl.kernel,
        mesh=mesh,
        compiler_params=pltpu.CompilerParams(use_tc_tiling_on_sc=False),
        out_type=jax.ShapeDtypeStruct((seq_len, half, 2 * _EMBED), jnp.float32),
        scratch_types=[
            pltpu.VMEM((8, 1024), jnp.int32),
            pltpu.VMEM((2, 2 * _RUN), jnp.int32),
            pltpu.VMEM((2, 2, _RUN, 2 * _EMBED), jnp.float32),
            pltpu.SemaphoreType.DMA,
            pltpu.SemaphoreType.DMA,
        ],
    )
    def sc_gather(xt_hbm, table_hbm, out_hbm, tok_v, keys_v, rows_v,
                  sem0, sem1):
        wid = lax.axis_index("s") * 2 + lax.axis_index("c")
        sems = (sem0, sem1)
        l0 = (wid * upw) // nhb
        lstart = jnp.maximum(l0 - 1, 0)
        pltpu.sync_copy(xt_hbm.at[pl.ds(lstart, 8)], tok_v)

        def unit(i):
            u = wid * upw + i
            return u // nhb, (u % nhb) * _RUN

        def stage_fire(i, slot):
            l, bb = unit(i)
            li = l - lstart
            lp = jnp.maximum(li - 1, 0)
            for side in range(2):
                off = side * half
                for j in range(_RUN // _LANES):
                    cur = tok_v[li, pl.ds(off + bb + j * _LANES, _LANES)]
                    prv = tok_v[lp, pl.ds(off + bb + j * _LANES, _LANES)]
                    keys_v[slot, pl.ds(side * _RUN + j * _LANES, _LANES)] = (
                        jnp.where(l > 0, prv * _VOCAB + cur, cur))
            pltpu.async_copy(table_hbm.at[keys_v.at[slot, pl.ds(0, _RUN)]],
                             rows_v.at[slot, 0], sems[slot])
            pltpu.async_copy(table_hbm.at[keys_v.at[slot, pl.ds(_RUN, _RUN)]],
                             rows_v.at[slot, 1], sems[slot])

        def drain(i, slot):
            pltpu.make_async_copy(
                table_hbm.at[keys_v.at[slot, pl.ds(0, _RUN)]],
                rows_v.at[slot, 0], sems[slot]).wait()
            pltpu.make_async_copy(
                table_hbm.at[keys_v.at[slot, pl.ds(_RUN, _RUN)]],
                rows_v.at[slot, 1], sems[slot]).wait()

        def scat(i, slot):
            l, bb = unit(i)
            pltpu.sync_copy(rows_v.at[slot, 0, slice(None), pl.ds(0, _EMBED)],
                            out_hbm.at[l, pl.ds(bb, _RUN), pl.ds(0, _EMBED)])
            pltpu.sync_copy(rows_v.at[slot, 1, slice(None), pl.ds(0, _EMBED)],
                            out_hbm.at[l, pl.ds(bb, _RUN), pl.ds(_EMBED, _EMBED)])

        stage_fire(0, 0)

        def body(i2, carry):
            k0 = i2 * 2
            stage_fire(k0 + 1, 1)
            drain(k0, 0)
            scat(k0, 0)

            @pl.when(k0 + 2 < upw)
            def _():
                stage_fire(k0 + 2, 0)

            drain(k0 + 1, 1)
            scat(k0 + 1, 1)
            return carry

        lax.fori_loop(0, upw // 2, body, 0)

    return sc_gather


_TCB = 8192


def _tpad_body(tin_ref, tout_ref):
    t = jnp.swapaxes(tin_ref[...], 0, 1)
    tout_ref[...] = jnp.concatenate(
        [t, jnp.zeros((_TCB, _EMBED), jnp.float32)], axis=1)


def _transpose_pad_table(table_t):
    nrows = table_t.shape[1]
    return pl.pallas_call(
        _tpad_body,
        grid=((nrows + _TCB - 1) // _TCB,),
        in_specs=[pl.BlockSpec((_EMBED, _TCB), lambda i: (0, i))],
        out_specs=pl.BlockSpec((_TCB, 2 * _EMBED), lambda i: (i, 0)),
        out_shape=jax.ShapeDtypeStruct((nrows, 2 * _EMBED), jnp.float32),
    )(table_t)


def _dense_body(h_ref, m_ref, gw_ref, gb_ref, ow_ref, ob_ref, out_ref, gate_ref):
    h = h_ref[0]
    mp = m_ref[0]
    dn = (((0,), (1,)), ((), ()))
    yl = lax.dot_general(ow_ref[...], mp[:, :_EMBED], dn,
                         preferred_element_type=jnp.float32)
    yr = lax.dot_general(ow_ref[...], mp[:, _EMBED:], dn,
                         preferred_element_type=jnp.float32)
    y = jnp.concatenate([yl, yr], axis=1) + ob_ref[...]
    g = jax.nn.sigmoid(
        jnp.sum(h * gw_ref[...], axis=0, keepdims=True) + gb_ref[0, 0])
    out_ref[0] = h + g * y
    gate_ref[0] = g


def kernel(x_tokens, hidden_state, memory_table, gate_w, gate_b, out_w, out_b):
    b, l = x_tokens.shape
    x_t = jnp.transpose(x_tokens).astype(jnp.int32)
    table128 = _transpose_pad_table(jnp.transpose(memory_table))
    gathered_t = _make_sc_gather(l, b)(x_t, table128)

    h_t = jnp.transpose(hidden_state, (1, 2, 0))
    out_t, gate_t = pl.pallas_call(
        _dense_body,
        grid=(l,),
        in_specs=[
            pl.BlockSpec((1, _EMBED, b), lambda i: (i, 0, 0)),
            pl.BlockSpec((1, b // 2, 2 * _EMBED), lambda i: (i, 0, 0)),
            pl.BlockSpec((_EMBED, 1), lambda i: (0, 0)),
            pl.BlockSpec((1, 1), lambda i: (0, 0)),
            pl.BlockSpec((_EMBED, _EMBED), lambda i: (0, 0)),
            pl.BlockSpec((_EMBED, 1), lambda i: (0, 0)),
        ],
        out_specs=[
            pl.BlockSpec((1, _EMBED, b), lambda i: (i, 0, 0)),
            pl.BlockSpec((1, 1, b), lambda i: (i, 0, 0)),
        ],
        out_shape=[
            jax.ShapeDtypeStruct((l, _EMBED, b), jnp.float32),
            jax.ShapeDtypeStruct((l, 1, b), jnp.float32),
        ],
    )(h_t, gathered_t, gate_w, gate_b.reshape(1, 1), out_w,
      out_b.reshape(_EMBED, 1))

    return (jnp.transpose(out_t, (2, 0, 1)),
            jnp.transpose(gate_t, (2, 0, 1)))


# speedup vs baseline: 3.2233x; 1.0397x over previous
import functools

import jax
import jax.numpy as jnp
from jax import lax
from jax.experimental import pallas as pl
from jax.experimental.pallas import tpu as pltpu
from jax.experimental.pallas import tpu_sc as plsc

_VOCAB = 1000
_EMBED = 64
_NW = 32
_LANES = 16
_RUN = 64


def _make_sc_gather(seq_len, batch):
    half = batch // 2
    nhb = half // _RUN
    n_units = seq_len * nhb
    upw = n_units // _NW
    mesh = plsc.VectorSubcoreMesh(core_axis_name="c", subcore_axis_name="s")

    @functools.partial(
        pl.kernel,
        mesh=mesh,
        compiler_params=pltpu.CompilerParams(use_tc_tiling_on_sc=False),
        out_type=jax.ShapeDtypeStruct((seq_len, half, 2 * _EMBED), jnp.float32),
        scratch_types=[
            pltpu.VMEM((8, 1024), jnp.int32),
            pltpu.VMEM((2, 2 * _RUN), jnp.int32),
            pltpu.VMEM((2, 2, _RUN, 2 * _EMBED), jnp.float32),
            pltpu.SemaphoreType.DMA,
            pltpu.SemaphoreType.DMA,
        ],
    )
    def sc_gather(xt_hbm, table_hbm, out_hbm, tok_v, keys_v, rows_v,
                  sem0, sem1):
        wid = lax.axis_index("s") * 2 + lax.axis_index("c")
        sems = (sem0, sem1)
        l0 = (wid * upw) // nhb
        lstart = jnp.maximum(l0 - 1, 0)
        pltpu.sync_copy(xt_hbm.at[pl.ds(lstart, 8)], tok_v)

        def unit(i):
            u = wid * upw + i
            return u // nhb, (u % nhb) * _RUN

        def stage_fire(i, slot):
            l, bb = unit(i)
            li = l - lstart
            lp = jnp.maximum(li - 1, 0)
            for side in range(2):
                off = side * half
                for j in range(_RUN // _LANES):
                    cur = tok_v[li, pl.ds(off + bb + j * _LANES, _LANES)]
                    prv = tok_v[lp, pl.ds(off + bb + j * _LANES, _LANES)]
                    keys_v[slot, pl.ds(side * _RUN + j * _LANES, _LANES)] = (
                        jnp.where(l > 0, prv * _VOCAB + cur, cur))
            pltpu.async_copy(table_hbm.at[keys_v.at[slot, pl.ds(0, _RUN)]],
                             rows_v.at[slot, 0], sems[slot])
            pltpu.async_copy(table_hbm.at[keys_v.at[slot, pl.ds(_RUN, _RUN)]],
                             rows_v.at[slot, 1], sems[slot])

        def drain(i, slot):
            pltpu.make_async_copy(
                table_hbm.at[keys_v.at[slot, pl.ds(0, _RUN)]],
                rows_v.at[slot, 0], sems[slot]).wait()
            pltpu.make_async_copy(
                table_hbm.at[keys_v.at[slot, pl.ds(_RUN, _RUN)]],
                rows_v.at[slot, 1], sems[slot]).wait()

        def scat(i, slot):
            l, bb = unit(i)
            pltpu.sync_copy(rows_v.at[slot, 0, slice(None), pl.ds(0, _EMBED)],
                            out_hbm.at[l, pl.ds(bb, _RUN), pl.ds(0, _EMBED)])
            pltpu.sync_copy(rows_v.at[slot, 1, slice(None), pl.ds(0, _EMBED)],
                            out_hbm.at[l, pl.ds(bb, _RUN), pl.ds(_EMBED, _EMBED)])

        stage_fire(0, 0)

        def body(i2, carry):
            k0 = i2 * 2
            stage_fire(k0 + 1, 1)
            drain(k0, 0)
            scat(k0, 0)

            @pl.when(k0 + 2 < upw)
            def _():
                stage_fire(k0 + 2, 0)

            drain(k0 + 1, 1)
            scat(k0 + 1, 1)
            return carry

        lax.fori_loop(0, upw // 2, body, 0)

    return sc_gather


_TCB = 16384


def _tpad_body(tin_ref, tout_ref):
    t = jnp.swapaxes(tin_ref[...], 0, 1)
    tout_ref[...] = jnp.concatenate(
        [t, jnp.zeros((_TCB, _EMBED), jnp.float32)], axis=1)


def _transpose_pad_table(table_t):
    nrows = table_t.shape[1]
    return pl.pallas_call(
        _tpad_body,
        grid=((nrows + _TCB - 1) // _TCB,),
        in_specs=[pl.BlockSpec((_EMBED, _TCB), lambda i: (0, i))],
        out_specs=pl.BlockSpec((_TCB, 2 * _EMBED), lambda i: (i, 0)),
        out_shape=jax.ShapeDtypeStruct((nrows, 2 * _EMBED), jnp.float32),
    )(table_t)


def _dense_body(h_ref, m_ref, gw_ref, gb_ref, ow_ref, ob_ref, out_ref, gate_ref):
    h = h_ref[0]
    mp = m_ref[0]
    dn = (((0,), (1,)), ((), ()))
    yl = lax.dot_general(ow_ref[...], mp[:, :_EMBED], dn,
                         preferred_element_type=jnp.float32)
    yr = lax.dot_general(ow_ref[...], mp[:, _EMBED:], dn,
                         preferred_element_type=jnp.float32)
    y = jnp.concatenate([yl, yr], axis=1) + ob_ref[...]
    g = jax.nn.sigmoid(
        jnp.sum(h * gw_ref[...], axis=0, keepdims=True) + gb_ref[0, 0])
    out_ref[0] = h + g * y
    gate_ref[0] = g


def kernel(x_tokens, hidden_state, memory_table, gate_w, gate_b, out_w, out_b):
    b, l = x_tokens.shape
    x_t = jnp.transpose(x_tokens).astype(jnp.int32)
    table128 = _transpose_pad_table(jnp.transpose(memory_table))
    gathered_t = _make_sc_gather(l, b)(x_t, table128)

    h_t = jnp.transpose(hidden_state, (1, 2, 0))
    out_t, gate_t = pl.pallas_call(
        _dense_body,
        grid=(l,),
        in_specs=[
            pl.BlockSpec((1, _EMBED, b), lambda i: (i, 0, 0)),
            pl.BlockSpec((1, b // 2, 2 * _EMBED), lambda i: (i, 0, 0)),
            pl.BlockSpec((_EMBED, 1), lambda i: (0, 0)),
            pl.BlockSpec((1, 1), lambda i: (0, 0)),
            pl.BlockSpec((_EMBED, _EMBED), lambda i: (0, 0)),
            pl.BlockSpec((_EMBED, 1), lambda i: (0, 0)),
        ],
        out_specs=[
            pl.BlockSpec((1, _EMBED, b), lambda i: (i, 0, 0)),
            pl.BlockSpec((1, 1, b), lambda i: (i, 0, 0)),
        ],
        out_shape=[
            jax.ShapeDtypeStruct((l, _EMBED, b), jnp.float32),
            jax.ShapeDtypeStruct((l, 1, b), jnp.float32),
        ],
    )(h_t, gathered_t, gate_w, gate_b.reshape(1, 1), out_w,
      out_b.reshape(_EMBED, 1))

    return (jnp.transpose(out_t, (2, 0, 1)),
            jnp.transpose(gate_t, (2, 0, 1)))


# speedup vs baseline: 3.2392x; 1.0049x over previous
import functools

import jax
import jax.numpy as jnp
from jax import lax
from jax.experimental import pallas as pl
from jax.experimental.pallas import tpu as pltpu
from jax.experimental.pallas import tpu_sc as plsc

_VOCAB = 1000
_EMBED = 64
_NW = 32
_LANES = 16
_RUN = 64


def _make_sc_gather(seq_len, batch):
    half = batch // 2
    nhb = half // _RUN
    n_units = seq_len * nhb
    upw = n_units // _NW
    mesh = plsc.VectorSubcoreMesh(core_axis_name="c", subcore_axis_name="s")

    @functools.partial(
        pl.kernel,
        mesh=mesh,
        compiler_params=pltpu.CompilerParams(use_tc_tiling_on_sc=False),
        out_type=jax.ShapeDtypeStruct((seq_len, half, 2 * _EMBED), jnp.float32),
        scratch_types=[
            pltpu.VMEM((8, 1024), jnp.int32),
            pltpu.VMEM((2, 2 * _RUN), jnp.int32),
            pltpu.VMEM((2, 2, _RUN, 2 * _EMBED), jnp.float32),
            pltpu.SemaphoreType.DMA,
            pltpu.SemaphoreType.DMA,
        ],
    )
    def sc_gather(xt_hbm, table_hbm, out_hbm, tok_v, keys_v, rows_v,
                  sem0, sem1):
        wid = lax.axis_index("s") * 2 + lax.axis_index("c")
        sems = (sem0, sem1)
        l0 = (wid * upw) // nhb
        lstart = jnp.maximum(l0 - 1, 0)
        pltpu.sync_copy(xt_hbm.at[pl.ds(lstart, 8)], tok_v)

        def unit(i):
            u = wid * upw + i
            return u // nhb, (u % nhb) * _RUN

        def stage_fire(i, slot):
            l, bb = unit(i)
            li = l - lstart
            lp = jnp.maximum(li - 1, 0)
            for side in range(2):
                off = side * half
                for j in range(_RUN // _LANES):
                    cur = tok_v[li, pl.ds(off + bb + j * _LANES, _LANES)]
                    prv = tok_v[lp, pl.ds(off + bb + j * _LANES, _LANES)]
                    keys_v[slot, pl.ds(side * _RUN + j * _LANES, _LANES)] = (
                        jnp.where(l > 0, prv * _VOCAB + cur, cur))
            pltpu.async_copy(table_hbm.at[keys_v.at[slot, pl.ds(0, _RUN)]],
                             rows_v.at[slot, 0], sems[slot])
            pltpu.async_copy(table_hbm.at[keys_v.at[slot, pl.ds(_RUN, _RUN)]],
                             rows_v.at[slot, 1], sems[slot])

        def drain(i, slot):
            pltpu.make_async_copy(
                table_hbm.at[keys_v.at[slot, pl.ds(0, _RUN)]],
                rows_v.at[slot, 0], sems[slot]).wait()
            pltpu.make_async_copy(
                table_hbm.at[keys_v.at[slot, pl.ds(_RUN, _RUN)]],
                rows_v.at[slot, 1], sems[slot]).wait()

        def scat(i, slot):
            l, bb = unit(i)
            pltpu.sync_copy(rows_v.at[slot, 0, slice(None), pl.ds(0, _EMBED)],
                            out_hbm.at[l, pl.ds(bb, _RUN), pl.ds(0, _EMBED)])
            pltpu.sync_copy(rows_v.at[slot, 1, slice(None), pl.ds(0, _EMBED)],
                            out_hbm.at[l, pl.ds(bb, _RUN), pl.ds(_EMBED, _EMBED)])

        stage_fire(0, 0)

        def body(i2, carry):
            k0 = i2 * 2
            stage_fire(k0 + 1, 1)
            drain(k0, 0)
            scat(k0, 0)

            @pl.when(k0 + 2 < upw)
            def _():
                stage_fire(k0 + 2, 0)

            drain(k0 + 1, 1)
            scat(k0 + 1, 1)
            return carry

        lax.fori_loop(0, upw // 2, body, 0)

    return sc_gather


_TCB = 32768


def _tpad_body(tin_ref, tout_ref):
    t = jnp.swapaxes(tin_ref[...], 0, 1)
    tout_ref[...] = jnp.concatenate(
        [t, jnp.zeros((_TCB, _EMBED), jnp.float32)], axis=1)


def _transpose_pad_table(table_t):
    nrows = table_t.shape[1]
    return pl.pallas_call(
        _tpad_body,
        grid=((nrows + _TCB - 1) // _TCB,),
        in_specs=[pl.BlockSpec((_EMBED, _TCB), lambda i: (0, i))],
        out_specs=pl.BlockSpec((_TCB, 2 * _EMBED), lambda i: (i, 0)),
        out_shape=jax.ShapeDtypeStruct((nrows, 2 * _EMBED), jnp.float32),
    )(table_t)


def _dense_body(h_ref, m_ref, gw_ref, gb_ref, ow_ref, ob_ref, out_ref, gate_ref):
    h = h_ref[0]
    mp = m_ref[0]
    dn = (((0,), (1,)), ((), ()))
    yl = lax.dot_general(ow_ref[...], mp[:, :_EMBED], dn,
                         preferred_element_type=jnp.float32)
    yr = lax.dot_general(ow_ref[...], mp[:, _EMBED:], dn,
                         preferred_element_type=jnp.float32)
    y = jnp.concatenate([yl, yr], axis=1) + ob_ref[...]
    g = jax.nn.sigmoid(
        jnp.sum(h * gw_ref[...], axis=0, keepdims=True) + gb_ref[0, 0])
    out_ref[0] = h + g * y
    gate_ref[0] = g


def kernel(x_tokens, hidden_state, memory_table, gate_w, gate_b, out_w, out_b):
    b, l = x_tokens.shape
    x_t = jnp.transpose(x_tokens).astype(jnp.int32)
    table128 = _transpose_pad_table(jnp.transpose(memory_table))
    gathered_t = _make_sc_gather(l, b)(x_t, table128)

    h_t = jnp.transpose(hidden_state, (1, 2, 0))
    out_t, gate_t = pl.pallas_call(
        _dense_body,
        grid=(l,),
        in_specs=[
            pl.BlockSpec((1, _EMBED, b), lambda i: (i, 0, 0)),
            pl.BlockSpec((1, b // 2, 2 * _EMBED), lambda i: (i, 0, 0)),
            pl.BlockSpec((_EMBED, 1), lambda i: (0, 0)),
            pl.BlockSpec((1, 1), lambda i: (0, 0)),
            pl.BlockSpec((_EMBED, _EMBED), lambda i: (0, 0)),
            pl.BlockSpec((_EMBED, 1), lambda i: (0, 0)),
        ],
        out_specs=[
            pl.BlockSpec((1, _EMBED, b), lambda i: (i, 0, 0)),
            pl.BlockSpec((1, 1, b), lambda i: (i, 0, 0)),
        ],
        out_shape=[
            jax.ShapeDtypeStruct((l, _EMBED, b), jnp.float32),
            jax.ShapeDtypeStruct((l, 1, b), jnp.float32),
        ],
    )(h_t, gathered_t, gate_w, gate_b.reshape(1, 1), out_w,
      out_b.reshape(_EMBED, 1))

    return (jnp.transpose(out_t, (2, 0, 1)),
            jnp.transpose(gate_t, (2, 0, 1)))


# speedup vs baseline: 3.6711x; 1.1333x over previous
import functools

import jax
import jax.numpy as jnp
from jax import lax
from jax.experimental import pallas as pl
from jax.experimental.pallas import tpu as pltpu
from jax.experimental.pallas import tpu_sc as plsc

_VOCAB = 1000
_EMBED = 64
_NW = 32
_LANES = 16
_RUN = 64


def _make_sc_gather(seq_len, batch):
    half = batch // 2
    nhb = half // _RUN
    n_units = seq_len * nhb
    upw = n_units // _NW
    mesh = plsc.VectorSubcoreMesh(core_axis_name="c", subcore_axis_name="s")

    @functools.partial(
        pl.kernel,
        mesh=mesh,
        compiler_params=pltpu.CompilerParams(use_tc_tiling_on_sc=False),
        out_type=jax.ShapeDtypeStruct((seq_len, half, 2 * _EMBED), jnp.float32),
        scratch_types=[
            pltpu.VMEM((8, 1024), jnp.int32),
            pltpu.VMEM((2, 2 * _RUN), jnp.int32),
            pltpu.VMEM((2, 2, _RUN, 2 * _EMBED), jnp.float32),
            pltpu.SemaphoreType.DMA,
            pltpu.SemaphoreType.DMA,
        ],
    )
    def sc_gather(xt_hbm, table_hbm, out_hbm, tok_v, keys_v, rows_v,
                  sem0, sem1):
        wid = lax.axis_index("s") * 2 + lax.axis_index("c")
        sems = (sem0, sem1)
        l0 = (wid * upw) // nhb
        lstart = jnp.maximum(l0 - 1, 0)
        pltpu.sync_copy(xt_hbm.at[pl.ds(lstart, 8)], tok_v)

        def unit(i):
            u = wid * upw + i
            return u // nhb, (u % nhb) * _RUN

        def stage_fire(i, slot):
            l, bb = unit(i)
            li = l - lstart
            lp = jnp.maximum(li - 1, 0)
            for side in range(2):
                off = side * half
                for j in range(_RUN // _LANES):
                    cur = tok_v[li, pl.ds(off + bb + j * _LANES, _LANES)]
                    prv = tok_v[lp, pl.ds(off + bb + j * _LANES, _LANES)]
                    keys_v[slot, pl.ds(side * _RUN + j * _LANES, _LANES)] = (
                        jnp.where(l > 0, prv * _VOCAB + cur, cur))
            pltpu.async_copy(table_hbm.at[keys_v.at[slot, pl.ds(0, _RUN)]],
                             rows_v.at[slot, 0], sems[slot])
            pltpu.async_copy(table_hbm.at[keys_v.at[slot, pl.ds(_RUN, _RUN)]],
                             rows_v.at[slot, 1], sems[slot])

        def drain(i, slot):
            pltpu.make_async_copy(
                table_hbm.at[keys_v.at[slot, pl.ds(0, _RUN)]],
                rows_v.at[slot, 0], sems[slot]).wait()
            pltpu.make_async_copy(
                table_hbm.at[keys_v.at[slot, pl.ds(_RUN, _RUN)]],
                rows_v.at[slot, 1], sems[slot]).wait()

        def scat(i, slot):
            l, bb = unit(i)
            pltpu.sync_copy(rows_v.at[slot, 0, slice(None), pl.ds(0, _EMBED)],
                            out_hbm.at[l, pl.ds(bb, _RUN), pl.ds(0, _EMBED)])
            pltpu.sync_copy(rows_v.at[slot, 1, slice(None), pl.ds(0, _EMBED)],
                            out_hbm.at[l, pl.ds(bb, _RUN), pl.ds(_EMBED, _EMBED)])

        stage_fire(0, 0)

        def body(i2, carry):
            k0 = i2 * 2
            stage_fire(k0 + 1, 1)
            drain(k0, 0)
            scat(k0, 0)

            @pl.when(k0 + 2 < upw)
            def _():
                stage_fire(k0 + 2, 0)

            drain(k0 + 1, 1)
            scat(k0 + 1, 1)
            return carry

        lax.fori_loop(0, upw // 2, body, 0)

    return sc_gather


_TCB = 32768


def _tpad_body(tin_ref, tout_ref):
    t = jnp.swapaxes(tin_ref[...], 0, 1)
    tout_ref[...] = jnp.concatenate(
        [t, jnp.zeros((_TCB, _EMBED), jnp.float32)], axis=1)


def _transpose_pad_table(table_t):
    nrows = table_t.shape[1]
    return pl.pallas_call(
        _tpad_body,
        grid=((nrows + _TCB - 1) // _TCB,),
        in_specs=[pl.BlockSpec((_EMBED, _TCB), lambda i: (0, i))],
        out_specs=pl.BlockSpec((_TCB, 2 * _EMBED), lambda i: (i, 0)),
        out_shape=jax.ShapeDtypeStruct((nrows, 2 * _EMBED), jnp.float32),
    )(table_t)


_LB = 2


def _dense_body(h_ref, m_ref, gw_ref, gb_ref, ow_ref, ob_ref, out_ref, gate_ref):
    dn = (((0,), (1,)), ((), ()))
    for s in range(_LB):
        h = h_ref[s]
        mp = m_ref[s]
        yl = lax.dot_general(ow_ref[...], mp[:, :_EMBED], dn,
                             preferred_element_type=jnp.float32)
        yr = lax.dot_general(ow_ref[...], mp[:, _EMBED:], dn,
                             preferred_element_type=jnp.float32)
        y = jnp.concatenate([yl, yr], axis=1) + ob_ref[...]
        g = jax.nn.sigmoid(
            jnp.sum(h * gw_ref[...], axis=0, keepdims=True) + gb_ref[0, 0])
        out_ref[s] = h + g * y
        gate_ref[s] = g


def kernel(x_tokens, hidden_state, memory_table, gate_w, gate_b, out_w, out_b):
    b, l = x_tokens.shape
    x_t = jnp.transpose(x_tokens).astype(jnp.int32)
    table128 = _transpose_pad_table(jnp.transpose(memory_table))
    gathered_t = _make_sc_gather(l, b)(x_t, table128)

    h_t = jnp.transpose(hidden_state, (1, 2, 0))
    out_t, gate_t = pl.pallas_call(
        _dense_body,
        grid=(l // _LB,),
        in_specs=[
            pl.BlockSpec((_LB, _EMBED, b), lambda i: (i, 0, 0)),
            pl.BlockSpec((_LB, b // 2, 2 * _EMBED), lambda i: (i, 0, 0)),
            pl.BlockSpec((_EMBED, 1), lambda i: (0, 0)),
            pl.BlockSpec((1, 1), lambda i: (0, 0)),
            pl.BlockSpec((_EMBED, _EMBED), lambda i: (0, 0)),
            pl.BlockSpec((_EMBED, 1), lambda i: (0, 0)),
        ],
        out_specs=[
            pl.BlockSpec((_LB, _EMBED, b), lambda i: (i, 0, 0)),
            pl.BlockSpec((_LB, 1, b), lambda i: (i, 0, 0)),
        ],
        out_shape=[
            jax.ShapeDtypeStruct((l, _EMBED, b), jnp.float32),
            jax.ShapeDtypeStruct((l, 1, b), jnp.float32),
        ],
    )(h_t, gathered_t, gate_w, gate_b.reshape(1, 1), out_w,
      out_b.reshape(_EMBED, 1))

    return (jnp.transpose(out_t, (2, 0, 1)),
            jnp.transpose(gate_t, (2, 0, 1)))


# speedup vs baseline: 3.9484x; 1.0755x over previous
import functools

import jax
import jax.numpy as jnp
from jax import lax
from jax.experimental import pallas as pl
from jax.experimental.pallas import tpu as pltpu
from jax.experimental.pallas import tpu_sc as plsc

_VOCAB = 1000
_EMBED = 64
_NW = 32
_LANES = 16
_RUN = 64


def _make_sc_gather(seq_len, batch):
    half = batch // 2
    nhb = half // _RUN
    n_units = seq_len * nhb
    upw = n_units // _NW
    mesh = plsc.VectorSubcoreMesh(core_axis_name="c", subcore_axis_name="s")

    @functools.partial(
        pl.kernel,
        mesh=mesh,
        compiler_params=pltpu.CompilerParams(use_tc_tiling_on_sc=False),
        out_type=jax.ShapeDtypeStruct((seq_len, half, 2 * _EMBED), jnp.float32),
        scratch_types=[
            pltpu.VMEM((8, 1024), jnp.int32),
            pltpu.VMEM((2, 2 * _RUN), jnp.int32),
            pltpu.VMEM((2, 2, _RUN, 2 * _EMBED), jnp.float32),
            pltpu.SemaphoreType.DMA,
            pltpu.SemaphoreType.DMA,
        ],
    )
    def sc_gather(xt_hbm, table_hbm, out_hbm, tok_v, keys_v, rows_v,
                  sem0, sem1):
        wid = lax.axis_index("s") * 2 + lax.axis_index("c")
        sems = (sem0, sem1)
        l0 = (wid * upw) // nhb
        lstart = jnp.maximum(l0 - 1, 0)
        pltpu.sync_copy(xt_hbm.at[pl.ds(lstart, 8)], tok_v)

        def unit(i):
            u = wid * upw + i
            return u // nhb, (u % nhb) * _RUN

        def stage_fire(i, slot):
            l, bb = unit(i)
            li = l - lstart
            lp = jnp.maximum(li - 1, 0)
            for side in range(2):
                off = side * half
                for j in range(_RUN // _LANES):
                    cur = tok_v[li, pl.ds(off + bb + j * _LANES, _LANES)]
                    prv = tok_v[lp, pl.ds(off + bb + j * _LANES, _LANES)]
                    keys_v[slot, pl.ds(side * _RUN + j * _LANES, _LANES)] = (
                        jnp.where(l > 0, prv * _VOCAB + cur, cur))
            pltpu.async_copy(table_hbm.at[keys_v.at[slot, pl.ds(0, _RUN)]],
                             rows_v.at[slot, 0], sems[slot])
            pltpu.async_copy(table_hbm.at[keys_v.at[slot, pl.ds(_RUN, _RUN)]],
                             rows_v.at[slot, 1], sems[slot])

        def drain(i, slot):
            pltpu.make_async_copy(
                table_hbm.at[keys_v.at[slot, pl.ds(0, _RUN)]],
                rows_v.at[slot, 0], sems[slot]).wait()
            pltpu.make_async_copy(
                table_hbm.at[keys_v.at[slot, pl.ds(_RUN, _RUN)]],
                rows_v.at[slot, 1], sems[slot]).wait()

        def scat(i, slot):
            l, bb = unit(i)
            pltpu.sync_copy(rows_v.at[slot, 0, slice(None), pl.ds(0, _EMBED)],
                            out_hbm.at[l, pl.ds(bb, _RUN), pl.ds(0, _EMBED)])
            pltpu.sync_copy(rows_v.at[slot, 1, slice(None), pl.ds(0, _EMBED)],
                            out_hbm.at[l, pl.ds(bb, _RUN), pl.ds(_EMBED, _EMBED)])

        stage_fire(0, 0)

        def body(i2, carry):
            k0 = i2 * 2
            stage_fire(k0 + 1, 1)
            drain(k0, 0)
            scat(k0, 0)

            @pl.when(k0 + 2 < upw)
            def _():
                stage_fire(k0 + 2, 0)

            drain(k0 + 1, 1)
            scat(k0 + 1, 1)
            return carry

        lax.fori_loop(0, upw // 2, body, 0)

    return sc_gather


_TCB = 32768


def _tpad_body(tin_ref, tout_ref):
    t = jnp.swapaxes(tin_ref[...], 0, 1)
    tout_ref[...] = jnp.concatenate(
        [t, jnp.zeros((_TCB, _EMBED), jnp.float32)], axis=1)


def _transpose_pad_table(table_t):
    nrows = table_t.shape[1]
    return pl.pallas_call(
        _tpad_body,
        grid=((nrows + _TCB - 1) // _TCB,),
        in_specs=[pl.BlockSpec((_EMBED, _TCB), lambda i: (0, i))],
        out_specs=pl.BlockSpec((_TCB, 2 * _EMBED), lambda i: (i, 0)),
        out_shape=jax.ShapeDtypeStruct((nrows, 2 * _EMBED), jnp.float32),
    )(table_t)


_LB = 4


def _dense_body(h_ref, m_ref, gw_ref, gb_ref, ow_ref, ob_ref, out_ref, gate_ref):
    dn = (((0,), (1,)), ((), ()))
    for s in range(_LB):
        h = h_ref[s]
        mp = m_ref[s]
        yl = lax.dot_general(ow_ref[...], mp[:, :_EMBED], dn,
                             preferred_element_type=jnp.float32)
        yr = lax.dot_general(ow_ref[...], mp[:, _EMBED:], dn,
                             preferred_element_type=jnp.float32)
        y = jnp.concatenate([yl, yr], axis=1) + ob_ref[...]
        g = jax.nn.sigmoid(
            jnp.sum(h * gw_ref[...], axis=0, keepdims=True) + gb_ref[0, 0])
        out_ref[s] = h + g * y
        gate_ref[s] = g


def kernel(x_tokens, hidden_state, memory_table, gate_w, gate_b, out_w, out_b):
    b, l = x_tokens.shape
    x_t = jnp.transpose(x_tokens).astype(jnp.int32)
    table128 = _transpose_pad_table(jnp.transpose(memory_table))
    gathered_t = _make_sc_gather(l, b)(x_t, table128)

    h_t = jnp.transpose(hidden_state, (1, 2, 0))
    out_t, gate_t = pl.pallas_call(
        _dense_body,
        grid=(l // _LB,),
        in_specs=[
            pl.BlockSpec((_LB, _EMBED, b), lambda i: (i, 0, 0)),
            pl.BlockSpec((_LB, b // 2, 2 * _EMBED), lambda i: (i, 0, 0)),
            pl.BlockSpec((_EMBED, 1), lambda i: (0, 0)),
            pl.BlockSpec((1, 1), lambda i: (0, 0)),
            pl.BlockSpec((_EMBED, _EMBED), lambda i: (0, 0)),
            pl.BlockSpec((_EMBED, 1), lambda i: (0, 0)),
        ],
        out_specs=[
            pl.BlockSpec((_LB, _EMBED, b), lambda i: (i, 0, 0)),
            pl.BlockSpec((_LB, 1, b), lambda i: (i, 0, 0)),
        ],
        out_shape=[
            jax.ShapeDtypeStruct((l, _EMBED, b), jnp.float32),
            jax.ShapeDtypeStruct((l, 1, b), jnp.float32),
        ],
    )(h_t, gathered_t, gate_w, gate_b.reshape(1, 1), out_w,
      out_b.reshape(_EMBED, 1))

    return (jnp.transpose(out_t, (2, 0, 1)),
            jnp.transpose(gate_t, (2, 0, 1)))
